# Initial kernel scaffold; baseline (speedup 1.0000x reference)
import jax
import jax.numpy as jnp
from jax.experimental import pallas as pl


def kernel(wordIdx, charIdx, char_table, glove_table):
    raise NotImplementedError("write your pallas kernel here")


# speedup vs baseline: 4.8802x; 4.8802x over previous
import functools

import jax
import jax.numpy as jnp
from jax import lax
from jax.experimental import pallas as pl
from jax.experimental.pallas import tpu as pltpu
from jax.experimental.pallas import tpu_sc as plsc

_NC, _NS = 2, 16
_NW = _NC * _NS
_MESH = plsc.VectorSubcoreMesh(core_axis_name="c", subcore_axis_name="s")
_PARAMS = pltpu.CompilerParams(use_tc_tiling_on_sc=False)
_DGP = 56


def _worker_id():
    return lax.axis_index("s") * _NC + lax.axis_index("c")


def _char_gather(cidx, char_table, n_rows, DC):
    per = n_rows // _NW
    CH = 2048
    n_c = per // CH

    @functools.partial(
        pl.kernel,
        out_type=jax.ShapeDtypeStruct((n_rows, DC), jnp.float32),
        mesh=_MESH,
        scratch_types=[
            pltpu.VMEM((CH,), jnp.int32),
            pltpu.VMEM((CH, DC), jnp.float32),
            pltpu.SemaphoreType.DMA,
        ],
        compiler_params=_PARAMS,
    )
    def emb(cidx_hbm, ctab_hbm, cout_hbm, cidx_v, crow_v, sem):
        base = _worker_id() * per

        @pl.loop(0, n_c)
        def _chunk(i):
            off = base + i * CH
            pltpu.sync_copy(cidx_hbm.at[pl.ds(off, CH)], cidx_v)
            pltpu.async_copy(ctab_hbm.at[cidx_v], crow_v, sem).wait()
            pltpu.sync_copy(crow_v, cout_hbm.at[pl.ds(off, CH)])

    return emb(cidx, char_table)


def _word_gather(widx, glove_pad, n_rows, DG):
    per = n_rows // _NW
    CW = 800
    n_w = per // CW

    @functools.partial(
        pl.kernel,
        out_type=jax.ShapeDtypeStruct((n_rows, _DGP), jnp.float32),
        mesh=_MESH,
        scratch_types=[
            pltpu.VMEM((CW,), jnp.int32),
            pltpu.VMEM((CW, _DGP), jnp.float32),
            pltpu.SemaphoreType.DMA,
        ],
        compiler_params=_PARAMS,
    )
    def emb(widx_hbm, gtab_hbm, wout_hbm, widx_v, wrow_v, sem):
        base = _worker_id() * per

        @pl.loop(0, n_w)
        def _chunk(i):
            off = base + i * CW
            pltpu.sync_copy(widx_hbm.at[pl.ds(off, CW)], widx_v)
            pltpu.async_copy(gtab_hbm.at[widx_v], wrow_v, sem).wait()
            pltpu.sync_copy(wrow_v, wout_hbm.at[pl.ds(off, CW)])

    return emb(widx, glove_pad)


def kernel(wordIdx, charIdx, char_table, glove_table):
    B, S = wordIdx.shape
    W = charIdx.shape[2]
    NWORD = B * S
    NCHAR = B * S * W
    DC = char_table.shape[1]
    DG = glove_table.shape[1]

    widx = wordIdx.reshape(NWORD)
    cidx = charIdx.reshape(NCHAR)
    glove_pad = jnp.pad(glove_table, ((0, 0), (0, _DGP - DG)))

    cout = _char_gather(cidx, char_table, NCHAR, DC)
    wout56 = _word_gather(widx, glove_pad, NWORD, DG)
    wout = wout56[:, :DG]
    return (cout.reshape(B, S, W, DC), wout.reshape(B, S, DG))


# speedup vs baseline: 11.7154x; 2.4006x over previous
import functools

import jax
import jax.numpy as jnp
from jax import lax
from jax.experimental import pallas as pl
from jax.experimental.pallas import tpu as pltpu
from jax.experimental.pallas import tpu_sc as plsc

_NC, _NS = 2, 16
_NW = _NC * _NS
_MESH = plsc.VectorSubcoreMesh(core_axis_name="c", subcore_axis_name="s")
_PARAMS = pltpu.CompilerParams(use_tc_tiling_on_sc=False, needs_layout_passes=False)
_DGP = 56
_L = 16


def _worker_id():
    return lax.axis_index("s") * _NC + lax.axis_index("c")


def _char_gather(cidx_t, char_tab_t, S, W, B, DC):
    V = char_tab_t.shape[0] // DC
    n_slabs = S * W
    per = n_slabs // _NW
    CT, BT = DC // 8, B // 128
    slab_words = DC * B

    del slab_words

    @functools.partial(
        pl.kernel,
        out_type=jax.ShapeDtypeStruct((n_slabs, CT, BT, 8, 128), jnp.float32),
        mesh=_MESH,
        scratch_types=[
            pltpu.VMEM((DC * V,), jnp.float32),
            pltpu.VMEM((B,), jnp.int32),
            pltpu.VMEM((CT, BT, 8, 128), jnp.float32),
            pltpu.SemaphoreType.DMA,
        ],
        compiler_params=_PARAMS,
    )
    def emb(cidx_hbm, ctab_hbm, out_hbm, tab_v, idx_v, slab_v, sem):
        wid = _worker_id()
        pltpu.sync_copy(ctab_hbm, tab_v)

        @pl.loop(0, per)
        def _slab(k):
            slab = wid * per + k
            pltpu.sync_copy(cidx_hbm.at[slab], idx_v)

            @pl.loop(0, BT)
            def _btile(bt):
                for blc in range(128 // _L):
                    idx16 = idx_v[pl.ds(bt * 128 + blc * _L, _L)]
                    for c in range(DC):
                        rows = plsc.load_gather(
                            tab_v, [idx16 + jnp.int32(c * V)])
                        slab_v[c // 8, bt, c % 8, pl.ds(blc * _L, _L)] = rows

            pltpu.sync_copy(slab_v, out_hbm.at[slab])

    return emb(cidx_t, char_tab_t)


def _word_gather(widx, glove_pad, n_rows, DG):
    per = n_rows // _NW
    CW = 800
    n_w = per // CW

    @functools.partial(
        pl.kernel,
        out_type=jax.ShapeDtypeStruct((n_rows, _DGP), jnp.float32),
        mesh=_MESH,
        scratch_types=[
            pltpu.VMEM((CW,), jnp.int32),
            pltpu.VMEM((CW, _DGP), jnp.float32),
            pltpu.SemaphoreType.DMA,
        ],
        compiler_params=_PARAMS,
    )
    def emb(widx_hbm, gtab_hbm, wout_hbm, widx_v, wrow_v, sem):
        base = _worker_id() * per

        @pl.loop(0, n_w)
        def _chunk(i):
            off = base + i * CW
            pltpu.sync_copy(widx_hbm.at[pl.ds(off, CW)], widx_v)
            pltpu.async_copy(gtab_hbm.at[widx_v], wrow_v, sem).wait()
            pltpu.sync_copy(wrow_v, wout_hbm.at[pl.ds(off, CW)])

    return emb(widx, glove_pad)


def kernel(wordIdx, charIdx, char_table, glove_table):
    B, S = wordIdx.shape
    W = charIdx.shape[2]
    NWORD = B * S
    DC = char_table.shape[1]
    DG = glove_table.shape[1]

    widx = wordIdx.reshape(NWORD)
    glove_pad = jnp.pad(glove_table, ((0, 0), (0, _DGP - DG)))

    cidx_t = charIdx.transpose(1, 2, 0).reshape(S * W, B)
    ctab_t = char_table.T.reshape(-1)

    cout5 = _char_gather(cidx_t, ctab_t, S, W, B, DC)
    cout6 = cout5.reshape(S, W, DC // 8, B // 128, 8, 128)
    cout = cout6.transpose(3, 5, 0, 1, 2, 4).reshape(B, S, W, DC)

    wout56 = _word_gather(widx, glove_pad, NWORD, DG)
    wout = wout56[:, :DG]
    return (cout, wout.reshape(B, S, DG))


# speedup vs baseline: 17.3042x; 1.4771x over previous
import functools

import jax
import jax.numpy as jnp
from jax import lax
from jax.experimental import pallas as pl
from jax.experimental.pallas import tpu as pltpu
from jax.experimental.pallas import tpu_sc as plsc

_NC, _NS = 2, 16
_NW = _NC * _NS
_MESH = plsc.VectorSubcoreMesh(core_axis_name="c", subcore_axis_name="s")
_PARAMS = pltpu.CompilerParams(
    use_tc_tiling_on_sc=False, needs_layout_passes=False)
_DGP = 56
_L = 16


def _worker_id():
    return lax.axis_index("s") * _NC + lax.axis_index("c")


def _char_gather(cidx5, char_tab_t, S, W, B, DC, V):
    n_slabs = S * W
    per = n_slabs // _NW
    CT, BT = DC // 8, B // 128

    @functools.partial(
        pl.kernel,
        out_type=jax.ShapeDtypeStruct((n_slabs, CT, BT, 8, 128), jnp.float32),
        mesh=_MESH,
        scratch_types=[
            pltpu.VMEM((DC * V,), jnp.float32),
            pltpu.VMEM((BT, 128), jnp.int32),
            pltpu.VMEM((BT, 128), jnp.int32),
            pltpu.VMEM((CT, BT, 8, 128), jnp.float32),
            pltpu.VMEM((CT, BT, 8, 128), jnp.float32),
            pltpu.SemaphoreType.DMA,
            pltpu.SemaphoreType.DMA,
            pltpu.SemaphoreType.DMA,
            pltpu.SemaphoreType.DMA,
        ],
        compiler_params=_PARAMS,
    )
    def emb(cidx_hbm, ctab_hbm, out_hbm, tab_v, idx0, idx1,
            slab0, slab1, si0, si1, so0, so1):
        wid = _worker_id()
        base = wid * per
        pltpu.sync_copy(ctab_hbm, tab_v)

        def idx_src(j):
            slab = base + j
            s = slab // W
            wq = slab % W
            return cidx_hbm.at[s, wq // 8, :, wq % 8]

        def compute(idx_v, slab_v):
            @pl.loop(0, BT)
            def _btile(bt):
                for blc in range(128 // _L):
                    idx16 = idx_v[bt, pl.ds(blc * _L, _L)]
                    for c in range(DC):
                        rows = plsc.load_gather(
                            tab_v, [idx16 + jnp.int32(c * V)])
                        slab_v[c // 8, bt, c % 8, pl.ds(blc * _L, _L)] = rows

        pltpu.async_copy(idx_src(0), idx0, si0)
        pltpu.async_copy(idx_src(1), idx1, si1)

        @pl.loop(0, per, step=2)
        def _pair(k):
            pltpu.make_async_copy(idx_src(k), idx0, si0).wait()

            @pl.when(k > 0)
            def _():
                pltpu.make_async_copy(slab0, out_hbm.at[base], so0).wait()

            compute(idx0, slab0)
            pltpu.async_copy(slab0, out_hbm.at[base + k], so0)
            pltpu.async_copy(idx_src(jnp.minimum(k + 2, per - 1)), idx0, si0)

            pltpu.make_async_copy(idx_src(k), idx1, si1).wait()

            @pl.when(k > 0)
            def _():
                pltpu.make_async_copy(slab1, out_hbm.at[base], so1).wait()

            compute(idx1, slab1)
            pltpu.async_copy(slab1, out_hbm.at[base + k + 1], so1)
            pltpu.async_copy(idx_src(jnp.minimum(k + 3, per - 1)), idx1, si1)

        pltpu.make_async_copy(idx_src(0), idx0, si0).wait()
        pltpu.make_async_copy(idx_src(0), idx1, si1).wait()
        pltpu.make_async_copy(slab0, out_hbm.at[base], so0).wait()
        pltpu.make_async_copy(slab1, out_hbm.at[base], so1).wait()

    return emb(cidx5, char_tab_t)


def _word_gather(widx3, glove_pad, S, B, DG):
    ST, BT = S // 8, B // 128
    n_chunks = ST * BT
    R = 8 * 128

    @functools.partial(
        pl.kernel,
        out_type=jax.ShapeDtypeStruct((DG, ST, BT, 8, 128), jnp.float32),
        mesh=_MESH,
        scratch_types=[
            pltpu.VMEM((R,), jnp.int32),
            pltpu.VMEM((R, _DGP), jnp.float32),
            pltpu.VMEM((8, 128), jnp.float32),
            pltpu.SemaphoreType.DMA,
        ],
        compiler_params=_PARAMS,
    )
    def emb(widx_hbm, gtab_hbm, out_hbm, idx_v, rows_v, plane_v, sem):
        wid = _worker_id()
        nq = jnp.where(wid < n_chunks - 6 * _NW, 7, 6)

        @pl.loop(0, nq)
        def _chunk(i):
            q = wid + i * _NW
            st = q // BT
            bt = q % BT
            pltpu.sync_copy(widx_hbm.at[st, bt], idx_v)
            pltpu.async_copy(gtab_hbm.at[idx_v], rows_v, sem).wait()

            @pl.loop(0, DG)
            def _dplane(d):
                dv = jnp.full((_L,), 0, jnp.int32) + d
                for sl in range(8):
                    for blc in range(128 // _L):
                        rvec = lax.iota(jnp.int32, _L) \
                            + jnp.int32(sl * 128 + blc * _L)
                        vals = plsc.load_gather(rows_v, [rvec, dv])
                        plane_v[sl, pl.ds(blc * _L, _L)] = vals
                pltpu.sync_copy(plane_v, out_hbm.at[d, st, bt])

    return emb(widx3, glove_pad)


def kernel(wordIdx, charIdx, char_table, glove_table):
    B, S = wordIdx.shape
    W = charIdx.shape[2]
    DC = char_table.shape[1]
    DG = glove_table.shape[1]
    V = char_table.shape[0]

    cidx5 = (charIdx.transpose(1, 2, 0)
             .reshape(S, W // 8, 8, B // 128, 128)
             .transpose(0, 1, 3, 2, 4))
    widx3 = (wordIdx.transpose(1, 0)
             .reshape(S // 8, 8, B // 128, 128)
             .transpose(0, 2, 1, 3)
             .reshape(S // 8, B // 128, 8 * 128))
    ctab_t = char_table.T.reshape(-1)
    glove_pad = jnp.pad(glove_table, ((0, 0), (0, _DGP - DG)))

    cout5 = _char_gather(cidx5, ctab_t, S, W, B, DC, V)
    wout5 = _word_gather(widx3, glove_pad, S, B, DG)

    cout = (cout5.reshape(S, W, DC // 8, B // 128, 8, 128)
            .transpose(3, 5, 0, 1, 2, 4).reshape(B, S, W, DC))
    wout = (wout5.transpose(2, 4, 1, 3, 0).reshape(B, S, DG))
    return (cout, wout)


# speedup vs baseline: 21.8503x; 1.2627x over previous
import functools

import jax
import jax.numpy as jnp
from jax import lax
from jax.experimental import pallas as pl
from jax.experimental.pallas import tpu as pltpu
from jax.experimental.pallas import tpu_sc as plsc

_NC, _NS = 2, 16
_NW = _NC * _NS
_MESH = plsc.VectorSubcoreMesh(core_axis_name="c", subcore_axis_name="s")
_PARAMS = pltpu.CompilerParams(
    use_tc_tiling_on_sc=False, needs_layout_passes=False)
_DGP = 56
_L = 16


def _worker_id():
    return lax.axis_index("s") * _NC + lax.axis_index("c")


def _char_gather(cidx5, char_tab_t, S, W, B, DC, V):
    n_slabs = S * W
    per = n_slabs // _NW
    CT, BT = DC // 8, B // 128

    @functools.partial(
        pl.kernel,
        out_type=jax.ShapeDtypeStruct((n_slabs, CT, BT, 8, 128), jnp.float32),
        mesh=_MESH,
        scratch_types=[
            pltpu.VMEM((DC * V,), jnp.float32),
            pltpu.VMEM((BT, 128), jnp.int32),
            pltpu.VMEM((BT, 128), jnp.int32),
            pltpu.VMEM((CT, BT, 8, 128), jnp.float32),
            pltpu.VMEM((CT, BT, 8, 128), jnp.float32),
            pltpu.SemaphoreType.DMA,
            pltpu.SemaphoreType.DMA,
            pltpu.SemaphoreType.DMA,
            pltpu.SemaphoreType.DMA,
        ],
        compiler_params=_PARAMS,
    )
    def emb(cidx_hbm, ctab_hbm, out_hbm, tab_v, idx0, idx1,
            slab0, slab1, si0, si1, so0, so1):
        wid = _worker_id()
        base = wid * per
        pltpu.sync_copy(ctab_hbm, tab_v)

        def idx_src(j):
            slab = base + j
            s = slab // W
            wq = slab % W
            return cidx_hbm.at[s, wq // 8, :, wq % 8]

        def compute(idx_v, slab_v):
            @pl.loop(0, BT)
            def _btile(bt):
                for blc in range(128 // _L):
                    idx16 = idx_v[bt, pl.ds(blc * _L, _L)]
                    rows = [plsc.load_gather(tab_v, [idx16 + jnp.int32(c * V)])
                            for c in range(DC)]
                    for c in range(DC):
                        slab_v[c // 8, bt, c % 8,
                               pl.ds(blc * _L, _L)] = rows[c]

        pltpu.async_copy(idx_src(0), idx0, si0)
        pltpu.async_copy(idx_src(1), idx1, si1)

        @pl.loop(0, per, step=2)
        def _pair(k):
            pltpu.make_async_copy(idx_src(k), idx0, si0).wait()

            @pl.when(k > 0)
            def _():
                pltpu.make_async_copy(slab0, out_hbm.at[base], so0).wait()

            compute(idx0, slab0)
            pltpu.async_copy(slab0, out_hbm.at[base + k], so0)
            pltpu.async_copy(idx_src(jnp.minimum(k + 2, per - 1)), idx0, si0)

            pltpu.make_async_copy(idx_src(k), idx1, si1).wait()

            @pl.when(k > 0)
            def _():
                pltpu.make_async_copy(slab1, out_hbm.at[base], so1).wait()

            compute(idx1, slab1)
            pltpu.async_copy(slab1, out_hbm.at[base + k + 1], so1)
            pltpu.async_copy(idx_src(jnp.minimum(k + 3, per - 1)), idx1, si1)

        pltpu.make_async_copy(idx_src(0), idx0, si0).wait()
        pltpu.make_async_copy(idx_src(0), idx1, si1).wait()
        pltpu.make_async_copy(slab0, out_hbm.at[base], so0).wait()
        pltpu.make_async_copy(slab1, out_hbm.at[base], so1).wait()

    return emb(cidx5, char_tab_t)


def _word_gather(widx3, glove_pad, S, B, DG):
    ST, BT = S // 8, B // 128
    n_chunks = ST * BT
    R = 8 * 128

    @functools.partial(
        pl.kernel,
        out_type=jax.ShapeDtypeStruct((DG, ST, BT, 8, 128), jnp.float32),
        mesh=_MESH,
        scratch_types=[
            pltpu.VMEM((R,), jnp.int32),
            pltpu.VMEM((R, _DGP), jnp.float32),
            pltpu.VMEM((DG, 8, 128), jnp.float32),
            pltpu.SemaphoreType.DMA,
            pltpu.SemaphoreType.DMA,
        ],
        compiler_params=_PARAMS,
    )
    def emb(widx_hbm, gtab_hbm, out_hbm, idx_v, rows_v, tile_v, sem, osem):
        wid = _worker_id()
        nq = jnp.where(wid < n_chunks - 6 * _NW, 7, 6)

        @pl.loop(0, nq)
        def _chunk(i):
            q = wid + i * _NW
            st = q // BT
            bt = q % BT
            pltpu.sync_copy(widx_hbm.at[st, bt], idx_v)
            pltpu.async_copy(gtab_hbm.at[idx_v], rows_v, sem).wait()

            @pl.loop(0, DG)
            def _dplane(d):
                dv = jnp.full((_L,), 0, jnp.int32) + d
                for sl in range(8):
                    vals = [plsc.load_gather(
                        rows_v,
                        [lax.iota(jnp.int32, _L)
                         + jnp.int32(sl * 128 + blc * _L), dv])
                        for blc in range(128 // _L)]
                    for blc in range(128 // _L):
                        tile_v[d, sl, pl.ds(blc * _L, _L)] = vals[blc]

            for d in range(DG):
                pltpu.async_copy(tile_v.at[d], out_hbm.at[d, st, bt], osem)
            for d in range(DG):
                pltpu.make_async_copy(
                    tile_v.at[d], out_hbm.at[0, st, bt], osem).wait()

    return emb(widx3, glove_pad)


def kernel(wordIdx, charIdx, char_table, glove_table):
    B, S = wordIdx.shape
    W = charIdx.shape[2]
    DC = char_table.shape[1]
    DG = glove_table.shape[1]
    V = char_table.shape[0]

    cidx5 = (charIdx.transpose(1, 2, 0)
             .reshape(S, W // 8, 8, B // 128, 128)
             .transpose(0, 1, 3, 2, 4))
    widx3 = (wordIdx.transpose(1, 0)
             .reshape(S // 8, 8, B // 128, 128)
             .transpose(0, 2, 1, 3)
             .reshape(S // 8, B // 128, 8 * 128))
    ctab_t = char_table.T.reshape(-1)
    glove_pad = jnp.pad(glove_table, ((0, 0), (0, _DGP - DG)))

    cout5 = _char_gather(cidx5, ctab_t, S, W, B, DC, V)
    wout5 = _word_gather(widx3, glove_pad, S, B, DG)

    cout = (cout5.reshape(S, W, DC // 8, B // 128, 8, 128)
            .transpose(3, 5, 0, 1, 2, 4).reshape(B, S, W, DC))
    wout = (wout5.transpose(2, 4, 1, 3, 0).reshape(B, S, DG))
    return (cout, wout)


# speedup vs baseline: 21.8790x; 1.0013x over previous
import functools

import jax
import jax.numpy as jnp
from jax import lax
from jax.experimental import pallas as pl
from jax.experimental.pallas import tpu as pltpu
from jax.experimental.pallas import tpu_sc as plsc

_NC, _NS = 2, 16
_NW = _NC * _NS
_MESH = plsc.VectorSubcoreMesh(core_axis_name="c", subcore_axis_name="s")
_PARAMS = pltpu.CompilerParams(
    use_tc_tiling_on_sc=False, needs_layout_passes=False)
_DGP = 56
_L = 16


def _worker_id():
    return lax.axis_index("s") * _NC + lax.axis_index("c")


def _char_gather(cidx5, char_tab_t, S, W, B, DC, V):
    n_slabs = S * W
    per = n_slabs // _NW
    CT, BT = DC // 8, B // 128

    @functools.partial(
        pl.kernel,
        out_type=jax.ShapeDtypeStruct((n_slabs, CT, BT, 8, 128), jnp.float32),
        mesh=_MESH,
        scratch_types=[
            pltpu.VMEM((DC * V,), jnp.float32),
            pltpu.VMEM((BT, 128), jnp.int32),
            pltpu.VMEM((BT, 128), jnp.int32),
            pltpu.VMEM((CT, BT, 8, 128), jnp.float32),
            pltpu.VMEM((CT, BT, 8, 128), jnp.float32),
            pltpu.SemaphoreType.DMA,
            pltpu.SemaphoreType.DMA,
            pltpu.SemaphoreType.DMA,
            pltpu.SemaphoreType.DMA,
        ],
        compiler_params=_PARAMS,
    )
    def emb(cidx_hbm, ctab_hbm, out_hbm, tab_v, idx0, idx1,
            slab0, slab1, si0, si1, so0, so1):
        wid = _worker_id()
        base = wid * per
        pltpu.sync_copy(ctab_hbm, tab_v)

        def idx_src(j):
            slab = base + j
            s = slab // W
            wq = slab % W
            return cidx_hbm.at[s, wq // 8, :, wq % 8]

        def compute(idx_v, slab_v):
            @pl.loop(0, BT)
            def _btile(bt):
                for blc in range(128 // _L):
                    idx16 = idx_v[bt, pl.ds(blc * _L, _L)]
                    rows = [plsc.load_gather(tab_v, [idx16 + jnp.int32(c * V)])
                            for c in range(DC)]
                    for c in range(DC):
                        slab_v[c // 8, bt, c % 8,
                               pl.ds(blc * _L, _L)] = rows[c]

        pltpu.async_copy(idx_src(0), idx0, si0)
        pltpu.async_copy(idx_src(1), idx1, si1)

        @pl.loop(0, per, step=2)
        def _pair(k):
            pltpu.make_async_copy(idx_src(k), idx0, si0).wait()

            @pl.when(k > 0)
            def _():
                pltpu.make_async_copy(slab0, out_hbm.at[base], so0).wait()

            compute(idx0, slab0)
            pltpu.async_copy(slab0, out_hbm.at[base + k], so0)
            pltpu.async_copy(idx_src(jnp.minimum(k + 2, per - 1)), idx0, si0)

            pltpu.make_async_copy(idx_src(k), idx1, si1).wait()

            @pl.when(k > 0)
            def _():
                pltpu.make_async_copy(slab1, out_hbm.at[base], so1).wait()

            compute(idx1, slab1)
            pltpu.async_copy(slab1, out_hbm.at[base + k + 1], so1)
            pltpu.async_copy(idx_src(jnp.minimum(k + 3, per - 1)), idx1, si1)

        pltpu.make_async_copy(idx_src(0), idx0, si0).wait()
        pltpu.make_async_copy(idx_src(0), idx1, si1).wait()
        pltpu.make_async_copy(slab0, out_hbm.at[base], so0).wait()
        pltpu.make_async_copy(slab1, out_hbm.at[base], so1).wait()

    return emb(cidx5, char_tab_t)


def _word_gather(widx3, glove_pad, S, B, DG):
    ST, BT = S // 8, B // 128
    n_chunks = ST * BT
    R = 8 * 128

    @functools.partial(
        pl.kernel,
        out_type=jax.ShapeDtypeStruct((DG, ST, BT, 8, 128), jnp.float32),
        mesh=_MESH,
        scratch_types=[
            pltpu.VMEM((R,), jnp.int32),
            pltpu.VMEM((R, _DGP), jnp.float32),
            pltpu.VMEM((DG, 8, 128), jnp.float32),
            pltpu.SemaphoreType.DMA,
            pltpu.SemaphoreType.DMA,
        ],
        compiler_params=_PARAMS,
    )
    def emb(widx_hbm, gtab_hbm, out_hbm, idx_v, rows_v, tile_v, sem, osem):
        wid = _worker_id()
        nq = jnp.where(wid < n_chunks - 6 * _NW, 7, 6)

        @pl.loop(0, nq)
        def _chunk(i):
            q = wid + i * _NW
            st = q // BT
            bt = q % BT
            pltpu.sync_copy(widx_hbm.at[st, bt], idx_v)
            pltpu.async_copy(gtab_hbm.at[idx_v], rows_v, sem).wait()

            @pl.loop(0, DG)
            def _dplane(d):
                dv = jnp.full((_L,), 0, jnp.int32) + d
                for sl in range(8):
                    vals = [plsc.load_gather(
                        rows_v,
                        [lax.iota(jnp.int32, _L)
                         + jnp.int32(sl * 128 + blc * _L), dv])
                        for blc in range(128 // _L)]
                    for blc in range(128 // _L):
                        tile_v[d, sl, pl.ds(blc * _L, _L)] = vals[blc]

            for d in range(DG):
                pltpu.async_copy(tile_v.at[d], out_hbm.at[d, st, bt], osem)
            for d in range(DG):
                pltpu.make_async_copy(
                    tile_v.at[d], out_hbm.at[0, st, bt], osem).wait()

    return emb(widx3, glove_pad)


def kernel(wordIdx, charIdx, char_table, glove_table):
    B, S = wordIdx.shape
    W = charIdx.shape[2]
    DC = char_table.shape[1]
    DG = glove_table.shape[1]
    V = char_table.shape[0]

    cidx5 = (charIdx.transpose(1, 2, 0)
             .reshape(S, W // 8, 8, B // 128, 128)
             .transpose(0, 1, 3, 2, 4))
    widx3 = (wordIdx.transpose(1, 0)
             .reshape(S // 8, 8, B // 128, 128)
             .transpose(0, 2, 1, 3)
             .reshape(S // 8, B // 128, 8 * 128))
    ctab_t = char_table.T.reshape(-1)
    glove_pad = jnp.pad(glove_table.T, ((0, _DGP - DG), (0, 0))).T

    cout5 = _char_gather(cidx5, ctab_t, S, W, B, DC, V)
    wout5 = _word_gather(widx3, glove_pad, S, B, DG)

    cout = (cout5.reshape(S, W, DC // 8, B // 128, 8, 128)
            .transpose(3, 5, 0, 1, 2, 4).reshape(B, S, W, DC))
    wout = (wout5.transpose(2, 4, 1, 3, 0).reshape(B, S, DG))
    return (cout, wout)
